# baseline (speedup 1.0000x reference)
import functools

import jax
import jax.numpy as jnp
from jax import lax
from jax.experimental import pallas as pl
from jax.experimental.pallas import tpu as pltpu
from jax.experimental.pallas import tpu_sc as plsc

MARGIN_ = 10.0
ALPHA_ = 1.0
EPS_ = 1e-05

BR = 2048
N_ = 32768
C_ = 1024
NW_ = 32
BPW_ = N_ // NW_


def _dense_body(x_ref, out_ref):
    i = pl.program_id(0)
    x = x_ref[...]

    m = jnp.max(x, axis=1, keepdims=True)
    d = x - m
    se = jnp.sum(jnp.exp(d), axis=1)
    s_lse = jnp.sum(m[:, 0] + jnp.log(se))
    s_relu = jnp.sum(jnp.maximum((-MARGIN_) - d, 0.0))
    s_x = jnp.sum(x)

    lane = jax.lax.broadcasted_iota(jnp.int32, (1, 128), 1)
    part = (jnp.where(lane == 0, s_lse, 0.0)
            + jnp.where(lane == 2, s_relu, 0.0)
            + jnp.where(lane == 3, s_x, 0.0))

    @pl.when(i == 0)
    def _():
        out_ref[...] = jnp.zeros_like(out_ref)

    out_ref[...] += part


def _dense_sums(inputs):
    n, c = inputs.shape
    grid = n // BR
    return pl.pallas_call(
        _dense_body,
        grid=(grid,),
        in_specs=[pl.BlockSpec((BR, c), lambda i: (i, 0))],
        out_specs=pl.BlockSpec((1, 128), lambda i: (0, 0)),
        out_shape=jax.ShapeDtypeStruct((1, 128), jnp.float32),
    )(inputs)


def _pick_body(x_hbm, t_hbm, out_hbm, t_v, idx_v, vals_v, acc_v, sem):
    wid = lax.axis_index("s") * 2 + lax.axis_index("c")
    base = wid * BPW_

    pltpu.sync_copy(t_hbm.at[pl.ds(base, BPW_)], t_v)

    lane16 = lax.iota(jnp.int32, 16)
    row0 = (base + lane16) * C_
    for k in range(8):
        for l in range(8):
            j = k * 8 + l
            t16 = t_v[pl.ds(j * 16, 16)]
            idx_v[k, pl.ds(l * 16, 16)] = t16 + row0 + (j * 16 * C_)

    copies = [pltpu.async_copy(x_hbm.at[idx_v.at[k]], vals_v.at[k], sem)
              for k in range(8)]
    for cp in copies:
        cp.wait()

    acc = jnp.zeros((16,), jnp.float32)
    for k in range(8):
        for l in range(8):
            acc = acc + vals_v[k, pl.ds(l * 16, 16)]
    acc_v[...] = acc
    pltpu.sync_copy(acc_v, out_hbm.at[wid])


@functools.partial(
    pl.kernel,
    mesh=plsc.VectorSubcoreMesh(core_axis_name="c", subcore_axis_name="s"),
    out_type=jax.ShapeDtypeStruct((NW_, 16), jnp.float32),
    scratch_types=[
        pltpu.VMEM((BPW_,), jnp.int32),
        pltpu.VMEM((8, 128), jnp.int32),
        pltpu.VMEM((8, 128), jnp.float32),
        pltpu.VMEM((16,), jnp.float32),
        pltpu.SemaphoreType.DMA,
    ],
)
def _pick_partials(x_flat_hbm, t_hbm, out_hbm, t_v, idx_v, vals_v, acc_v, sem):
    _pick_body(x_flat_hbm, t_hbm, out_hbm, t_v, idx_v, vals_v, acc_v, sem)


def kernel(inputs, targets):
    n, c = inputs.shape
    t32 = targets.astype(jnp.int32)

    pick_parts = _pick_partials(inputs.reshape(-1), t32)
    dense = _dense_sums(inputs)

    s_pick = jnp.sum(pick_parts)
    s_lse, s_relu, s_x = dense[0, 0], dense[0, 2], dense[0, 3]

    loss_ce = (s_lse - s_pick) / n
    loss_margin = s_relu / (n * c)
    dice = (2.0 * s_pick + EPS_) / ((n + s_x) + EPS_)
    loss_dice = 1.0 - dice
    loss = loss_ce + loss_dice + ALPHA_ * loss_margin
    return (loss, loss_ce, loss_margin, loss_dice)


# speedup vs baseline: 2.1770x; 2.1770x over previous
import functools

import jax
import jax.numpy as jnp
from jax import lax
from jax.experimental import pallas as pl
from jax.experimental.pallas import tpu as pltpu
from jax.experimental.pallas import tpu_sc as plsc

MARGIN_ = 10.0
ALPHA_ = 1.0
EPS_ = 1e-05

BR = 2048
N_ = 32768
C_ = 1024
NW_ = 32
BPW_ = N_ // NW_


def _dense_body(x_ref, t_ref, out_ref):
    i = pl.program_id(0)
    x = x_ref[...]
    t = t_ref[0, 0, :]
    br, c = x.shape

    m = jnp.max(x, axis=1, keepdims=True)
    d = x - m
    se = jnp.sum(jnp.exp(d), axis=1)
    s_lse = jnp.sum(m[:, 0] + jnp.log(se))
    s_relu = jnp.sum(jnp.maximum((-MARGIN_) - d, 0.0))
    s_x = jnp.sum(x)
    cols = jax.lax.broadcasted_iota(jnp.int32, (br, c), 1)
    s_pick = jnp.sum(jnp.where(cols == t[:, None], x, 0.0))

    lane = jax.lax.broadcasted_iota(jnp.int32, (1, 128), 1)
    part = (jnp.where(lane == 0, s_lse, 0.0)
            + jnp.where(lane == 1, s_pick, 0.0)
            + jnp.where(lane == 2, s_relu, 0.0)
            + jnp.where(lane == 3, s_x, 0.0))

    @pl.when(i == 0)
    def _():
        out_ref[...] = jnp.zeros_like(out_ref)

    out_ref[...] += part


def _dense_sums(inputs, t32):
    n, c = inputs.shape
    grid = n // BR
    t3 = t32.reshape(grid, 1, BR)
    return pl.pallas_call(
        _dense_body,
        grid=(grid,),
        in_specs=[pl.BlockSpec((BR, c), lambda i: (i, 0)),
                  pl.BlockSpec((1, 1, BR), lambda i: (i, 0, 0))],
        out_specs=pl.BlockSpec((1, 128), lambda i: (0, 0)),
        out_shape=jax.ShapeDtypeStruct((1, 128), jnp.float32),
    )(inputs, t3)


def _pick_body(x_hbm, t_hbm, out_hbm, t_v, idx_v, vals_v, acc_v, sem):
    wid = lax.axis_index("s") * 2 + lax.axis_index("c")
    base = wid * BPW_

    pltpu.sync_copy(t_hbm.at[pl.ds(base, BPW_)], t_v)

    lane16 = lax.iota(jnp.int32, 16)
    row0 = (base + lane16) * C_
    for k in range(8):
        for l in range(8):
            j = k * 8 + l
            t16 = t_v[pl.ds(j * 16, 16)]
            idx_v[k, pl.ds(l * 16, 16)] = t16 + row0 + (j * 16 * C_)

    copies = [pltpu.async_copy(x_hbm.at[idx_v.at[k]], vals_v.at[k], sem)
              for k in range(8)]
    for cp in copies:
        cp.wait()

    acc = jnp.zeros((16,), jnp.float32)
    for k in range(8):
        for l in range(8):
            acc = acc + vals_v[k, pl.ds(l * 16, 16)]
    acc_v[...] = acc
    pltpu.sync_copy(acc_v, out_hbm.at[wid])


@functools.partial(
    pl.kernel,
    mesh=plsc.VectorSubcoreMesh(core_axis_name="c", subcore_axis_name="s"),
    out_type=jax.ShapeDtypeStruct((NW_, 16), jnp.float32),
    scratch_types=[
        pltpu.VMEM((BPW_,), jnp.int32),
        pltpu.VMEM((8, 128), jnp.int32),
        pltpu.VMEM((8, 128), jnp.float32),
        pltpu.VMEM((16,), jnp.float32),
        pltpu.SemaphoreType.DMA,
    ],
)
def _pick_partials(x_flat_hbm, t_hbm, out_hbm, t_v, idx_v, vals_v, acc_v, sem):
    _pick_body(x_flat_hbm, t_hbm, out_hbm, t_v, idx_v, vals_v, acc_v, sem)


def kernel(inputs, targets):
    n, c = inputs.shape
    t32 = targets.astype(jnp.int32)

    dense = _dense_sums(inputs, t32)

    s_lse, s_pick = dense[0, 0], dense[0, 1]
    s_relu, s_x = dense[0, 2], dense[0, 3]

    loss_ce = (s_lse - s_pick) / n
    loss_margin = s_relu / (n * c)
    dice = (2.0 * s_pick + EPS_) / ((n + s_x) + EPS_)
    loss_dice = 1.0 - dice
    loss = loss_ce + loss_dice + ALPHA_ * loss_margin
    return (loss, loss_ce, loss_margin, loss_dice)


# speedup vs baseline: 2.3260x; 1.0685x over previous
import functools

import jax
import jax.numpy as jnp
from jax import lax
from jax.experimental import pallas as pl
from jax.experimental.pallas import tpu as pltpu
from jax.experimental.pallas import tpu_sc as plsc

MARGIN_ = 10.0
ALPHA_ = 1.0
EPS_ = 1e-05

BR = 2048
N_ = 32768
C_ = 1024
NW_ = 32
BPW_ = N_ // NW_


def _dense_body(x_ref, t_ref, out_ref):
    i = pl.program_id(0)
    x = x_ref[...]
    t = t_ref[0, 0, :]
    br, c = x.shape

    m = jnp.max(x, axis=1, keepdims=True)
    se = jnp.sum(jnp.exp(x - m), axis=1)
    s_lse = jnp.sum(m[:, 0] + jnp.log(se))
    s_relu = jnp.sum(jnp.maximum((m - MARGIN_) - x, 0.0))
    s_x = jnp.sum(x)
    cols = jax.lax.broadcasted_iota(jnp.int32, (br, c), 1)
    s_pick = jnp.sum(jnp.where(cols == t[:, None], x, 0.0))

    lane = jax.lax.broadcasted_iota(jnp.int32, (1, 128), 1)
    part = (jnp.where(lane == 0, s_lse, 0.0)
            + jnp.where(lane == 1, s_pick, 0.0)
            + jnp.where(lane == 2, s_relu, 0.0)
            + jnp.where(lane == 3, s_x, 0.0))

    @pl.when(i == 0)
    def _():
        out_ref[...] = jnp.zeros_like(out_ref)

    out_ref[...] += part


def _dense_sums(inputs, t32):
    n, c = inputs.shape
    grid = n // BR
    t3 = t32.reshape(grid, 1, BR)
    return pl.pallas_call(
        _dense_body,
        grid=(grid,),
        in_specs=[pl.BlockSpec((BR, c), lambda i: (i, 0)),
                  pl.BlockSpec((1, 1, BR), lambda i: (i, 0, 0))],
        out_specs=pl.BlockSpec((1, 128), lambda i: (0, 0)),
        out_shape=jax.ShapeDtypeStruct((1, 128), jnp.float32),
    )(inputs, t3)


def _pick_body(x_hbm, t_hbm, out_hbm, t_v, idx_v, vals_v, acc_v, sem):
    wid = lax.axis_index("s") * 2 + lax.axis_index("c")
    base = wid * BPW_

    pltpu.sync_copy(t_hbm.at[pl.ds(base, BPW_)], t_v)

    lane16 = lax.iota(jnp.int32, 16)
    row0 = (base + lane16) * C_
    for k in range(8):
        for l in range(8):
            j = k * 8 + l
            t16 = t_v[pl.ds(j * 16, 16)]
            idx_v[k, pl.ds(l * 16, 16)] = t16 + row0 + (j * 16 * C_)

    copies = [pltpu.async_copy(x_hbm.at[idx_v.at[k]], vals_v.at[k], sem)
              for k in range(8)]
    for cp in copies:
        cp.wait()

    acc = jnp.zeros((16,), jnp.float32)
    for k in range(8):
        for l in range(8):
            acc = acc + vals_v[k, pl.ds(l * 16, 16)]
    acc_v[...] = acc
    pltpu.sync_copy(acc_v, out_hbm.at[wid])


@functools.partial(
    pl.kernel,
    mesh=plsc.VectorSubcoreMesh(core_axis_name="c", subcore_axis_name="s"),
    out_type=jax.ShapeDtypeStruct((NW_, 16), jnp.float32),
    scratch_types=[
        pltpu.VMEM((BPW_,), jnp.int32),
        pltpu.VMEM((8, 128), jnp.int32),
        pltpu.VMEM((8, 128), jnp.float32),
        pltpu.VMEM((16,), jnp.float32),
        pltpu.SemaphoreType.DMA,
    ],
)
def _pick_partials(x_flat_hbm, t_hbm, out_hbm, t_v, idx_v, vals_v, acc_v, sem):
    _pick_body(x_flat_hbm, t_hbm, out_hbm, t_v, idx_v, vals_v, acc_v, sem)


def kernel(inputs, targets):
    n, c = inputs.shape
    t32 = targets.astype(jnp.int32)

    dense = _dense_sums(inputs, t32)

    s_lse, s_pick = dense[0, 0], dense[0, 1]
    s_relu, s_x = dense[0, 2], dense[0, 3]

    loss_ce = (s_lse - s_pick) / n
    loss_margin = s_relu / (n * c)
    dice = (2.0 * s_pick + EPS_) / ((n + s_x) + EPS_)
    loss_dice = 1.0 - dice
    loss = loss_ce + loss_dice + ALPHA_ * loss_margin
    return (loss, loss_ce, loss_margin, loss_dice)
